# Initial kernel scaffold; baseline (speedup 1.0000x reference)
import jax
import jax.numpy as jnp
from jax.experimental import pallas as pl


def kernel(hidden_states, residual, ln_weight, ln_bias, router_weight, gate_up_proj, down_proj):
    raise NotImplementedError("write your pallas kernel here")


# speedup vs baseline: 1.4792x; 1.4792x over previous
import functools

import jax
import jax.numpy as jnp
from jax import lax
from jax.experimental import pallas as pl

E = 8
TOPK = 2
D = 1024
DFF = 4096
T_BLK = 256
F_BLK = 512

_INTERPRET = False


def _ffn_body(x_ref, wg_ref, wu_ref, wd_ref, wt_ref, h_ref, out_ref):
    e = pl.program_id(0)
    k = pl.program_id(1)

    @pl.when(jnp.logical_and(e == 0, k == 0))
    def _init():
        out_ref[...] = h_ref[...]

    x = x_ref[...]
    wg = wg_ref[0].astype(jnp.bfloat16)
    wu = wu_ref[0].astype(jnp.bfloat16)
    g = lax.dot_general(x, wg, (((1,), (1,)), ((), ())),
                        preferred_element_type=jnp.float32)
    u = lax.dot_general(x, wu, (((1,), (1,)), ((), ())),
                        preferred_element_type=jnp.float32)
    act = (g * jax.nn.sigmoid(g)) * u
    w = wt_ref[0, 0, :]
    act_bf = (act * w[:, None]).astype(jnp.bfloat16)
    wd = wd_ref[0].astype(jnp.bfloat16)
    out_ref[...] += lax.dot_general(act_bf, wd, (((1,), (1,)), ((), ())),
                                    preferred_element_type=jnp.float32)


def kernel(hidden_states, residual, ln_weight, ln_bias, router_weight,
           gate_up_proj, down_proj):
    B, S, _ = hidden_states.shape
    T = B * S

    h3 = residual + hidden_states
    mu = jnp.mean(h3, axis=-1, keepdims=True)
    var = jnp.mean((h3 - mu) ** 2, axis=-1, keepdims=True)
    normed = (h3 - mu) * lax.rsqrt(var + 1e-5) * ln_weight + ln_bias
    hidden_2d = normed.reshape(T, D)
    logits = hidden_2d @ router_weight.T
    probs = jax.nn.softmax(logits, axis=-1)
    routing_weights, selected_experts = lax.top_k(probs, TOPK)
    routing_weights = routing_weights / jnp.sum(routing_weights, axis=-1,
                                                keepdims=True)
    onehot = jax.nn.one_hot(selected_experts, E, dtype=routing_weights.dtype)
    wfull = jnp.sum(routing_weights[..., None] * onehot, axis=1)

    h = h3.reshape(T, D)
    xbf = hidden_2d.astype(jnp.bfloat16)
    wt = wfull.T.reshape(E, 1, T)

    K = DFF // F_BLK
    out = pl.pallas_call(
        _ffn_body,
        grid=(E, K),
        in_specs=[
            pl.BlockSpec((T, D), lambda e, k: (0, 0)),
            pl.BlockSpec((1, F_BLK, D), lambda e, k: (e, k, 0)),
            pl.BlockSpec((1, F_BLK, D), lambda e, k: (e, K + k, 0)),
            pl.BlockSpec((1, D, F_BLK), lambda e, k: (e, 0, k)),
            pl.BlockSpec((1, 1, T), lambda e, k: (e, 0, 0)),
            pl.BlockSpec((T, D), lambda e, k: (0, 0)),
        ],
        out_specs=pl.BlockSpec((T, D), lambda e, k: (0, 0)),
        out_shape=jax.ShapeDtypeStruct((T, D), jnp.float32),
        interpret=_INTERPRET,
    )(xbf, gate_up_proj, gate_up_proj, down_proj, wt, h)

    return out.reshape(B, S, D)


# speedup vs baseline: 1.5371x; 1.0391x over previous
import functools

import jax
import jax.numpy as jnp
from jax import lax
from jax.experimental import pallas as pl
from jax.experimental.pallas import tpu as pltpu

E = 8
TOPK = 2
D = 1024
DFF = 4096
M = 256
NT = 23
NROWS = NT * M
F_BLK = 1024
K1 = DFF // F_BLK

_INTERPRET = False


def _gmm1_body(te_ref, x_ref, wg_ref, wu_ref, ws_ref, act_ref):
    x = x_ref[...]
    wg = wg_ref[0].astype(jnp.bfloat16)
    wu = wu_ref[0].astype(jnp.bfloat16)
    g = lax.dot_general(x, wg, (((1,), (1,)), ((), ())),
                        preferred_element_type=jnp.float32)
    u = lax.dot_general(x, wu, (((1,), (1,)), ((), ())),
                        preferred_element_type=jnp.float32)
    act = (g * jax.nn.sigmoid(g)) * u
    w = ws_ref[0, 0, :]
    act_ref[...] = (act * w[:, None]).astype(jnp.bfloat16)


def _gmm2_body(te_ref, act_ref, wd_ref, out_ref):
    act = act_ref[...]
    wd = wd_ref[0].astype(jnp.bfloat16)
    out_ref[...] = lax.dot_general(act, wd, (((1,), (1,)), ((), ())),
                                   preferred_element_type=jnp.float32)


def kernel(hidden_states, residual, ln_weight, ln_bias, router_weight,
           gate_up_proj, down_proj):
    B, S, _ = hidden_states.shape
    T = B * S

    h3 = residual + hidden_states
    mu = jnp.mean(h3, axis=-1, keepdims=True)
    var = jnp.mean((h3 - mu) ** 2, axis=-1, keepdims=True)
    normed = (h3 - mu) * lax.rsqrt(var + 1e-5) * ln_weight + ln_bias
    hidden_2d = normed.reshape(T, D)
    logits = hidden_2d @ router_weight.T
    probs = jax.nn.softmax(logits, axis=-1)
    routing_weights, selected_experts = lax.top_k(probs, TOPK)
    routing_weights = routing_weights / jnp.sum(routing_weights, axis=-1,
                                                keepdims=True)

    eflat = selected_experts.reshape(-1).astype(jnp.int32)
    rwflat = routing_weights.reshape(-1)
    oh = jax.nn.one_hot(eflat, E, dtype=jnp.int32)
    csum = jnp.cumsum(oh, axis=0)
    counts = csum[-1]
    rank = jnp.take_along_axis(csum, eflat[:, None], axis=1)[:, 0] - 1
    ntiles = (counts + M - 1) // M
    tile_end = jnp.cumsum(ntiles)
    tile_start = tile_end - ntiles
    row_start = tile_start * M
    pos = jnp.take(row_start, eflat) + rank
    gidx = jnp.zeros((NROWS,), jnp.int32).at[pos].set(
        jnp.arange(2 * T, dtype=jnp.int32) // TOPK)
    ws = jnp.zeros((NROWS,), jnp.float32).at[pos].set(rwflat)
    tile_ids = jnp.arange(NT, dtype=jnp.int32)
    te = jnp.searchsorted(tile_end, tile_ids, side='right').astype(jnp.int32)
    te = jnp.minimum(te, E - 1)

    xbf = hidden_2d.astype(jnp.bfloat16)
    x_sorted = jnp.take(xbf, gidx, axis=0)

    ws3 = ws.reshape(NT, 1, M)
    act = pl.pallas_call(
        _gmm1_body,
        grid_spec=pltpu.PrefetchScalarGridSpec(
            num_scalar_prefetch=1,
            grid=(K1, NT),
            in_specs=[
                pl.BlockSpec((M, D), lambda k, t, te: (t, 0)),
                pl.BlockSpec((1, F_BLK, D), lambda k, t, te: (te[t], k, 0)),
                pl.BlockSpec((1, F_BLK, D),
                             lambda k, t, te: (te[t], K1 + k, 0)),
                pl.BlockSpec((1, 1, M), lambda k, t, te: (t, 0, 0)),
            ],
            out_specs=pl.BlockSpec((M, F_BLK), lambda k, t, te: (t, k)),
        ),
        out_shape=jax.ShapeDtypeStruct((NROWS, DFF), jnp.bfloat16),
        interpret=_INTERPRET,
    )(te, x_sorted, gate_up_proj, gate_up_proj, ws3)

    y_sorted = pl.pallas_call(
        _gmm2_body,
        grid_spec=pltpu.PrefetchScalarGridSpec(
            num_scalar_prefetch=1,
            grid=(NT,),
            in_specs=[
                pl.BlockSpec((M, DFF), lambda t, te: (t, 0)),
                pl.BlockSpec((1, D, DFF), lambda t, te: (te[t], 0, 0)),
            ],
            out_specs=pl.BlockSpec((M, D), lambda t, te: (t, 0)),
        ),
        out_shape=jax.ShapeDtypeStruct((NROWS, D), jnp.float32),
        interpret=_INTERPRET,
    )(te, act, down_proj)

    pos2 = pos.reshape(T, TOPK)
    y1 = jnp.take(y_sorted, pos2[:, 0], axis=0)
    y2 = jnp.take(y_sorted, pos2[:, 1], axis=0)
    out = h3.reshape(T, D) + y1 + y2
    return out.reshape(B, S, D)


# speedup vs baseline: 1.5590x; 1.0143x over previous
import functools

import jax
import jax.numpy as jnp
from jax import lax
from jax.experimental import pallas as pl
from jax.experimental.pallas import tpu as pltpu

E = 8
TOPK = 2
D = 1024
DFF = 4096
M = 512
NT = 15
NROWS = NT * M
F_BLK = 1024
K1 = DFF // F_BLK

_INTERPRET = False


def _gmm1_body(te_ref, x_ref, wg_ref, wu_ref, ws_ref, act_ref):
    t = pl.program_id(1)
    x = x_ref[pl.ds(t * M, M), :]
    wg = wg_ref[0].astype(jnp.bfloat16)
    wu = wu_ref[0].astype(jnp.bfloat16)
    g = lax.dot_general(x, wg, (((1,), (1,)), ((), ())),
                        preferred_element_type=jnp.float32)
    u = lax.dot_general(x, wu, (((1,), (1,)), ((), ())),
                        preferred_element_type=jnp.float32)
    act = (g * jax.nn.sigmoid(g)) * u
    w = ws_ref[0, 0, :]
    act_ref[...] = (act * w[:, None]).astype(jnp.bfloat16)


def _gmm2_body(te_ref, act_ref, wd_ref, out_ref):
    act = act_ref[...]
    wd = wd_ref[0].astype(jnp.bfloat16)
    out_ref[...] = lax.dot_general(act, wd, (((1,), (1,)), ((), ())),
                                   preferred_element_type=jnp.float32)


def kernel(hidden_states, residual, ln_weight, ln_bias, router_weight,
           gate_up_proj, down_proj):
    B, S, _ = hidden_states.shape
    T = B * S

    h3 = residual + hidden_states
    mu = jnp.mean(h3, axis=-1, keepdims=True)
    var = jnp.mean((h3 - mu) ** 2, axis=-1, keepdims=True)
    normed = (h3 - mu) * lax.rsqrt(var + 1e-5) * ln_weight + ln_bias
    hidden_2d = normed.reshape(T, D)
    logits = hidden_2d @ router_weight.T
    probs = jax.nn.softmax(logits, axis=-1)
    routing_weights, selected_experts = lax.top_k(probs, TOPK)
    routing_weights = routing_weights / jnp.sum(routing_weights, axis=-1,
                                                keepdims=True)

    eflat = selected_experts.reshape(-1).astype(jnp.int32)
    rwflat = routing_weights.reshape(-1)
    oh = jax.nn.one_hot(eflat, E, dtype=jnp.int32)
    csum = jnp.cumsum(oh, axis=0)
    counts = csum[-1]
    rank = jnp.take_along_axis(csum, eflat[:, None], axis=1)[:, 0] - 1
    ntiles = (counts + M - 1) // M
    tile_end = jnp.cumsum(ntiles)
    tile_start = tile_end - ntiles
    row_start = tile_start * M
    pos = jnp.take(row_start, eflat) + rank
    tile_ids = jnp.arange(NT, dtype=jnp.int32)
    te = jnp.searchsorted(tile_end, tile_ids, side='right').astype(jnp.int32)
    te = jnp.minimum(te, E - 1)
    gidx = jnp.zeros((NROWS,), jnp.int32).at[pos].set(
        jnp.arange(2 * T, dtype=jnp.int32) // TOPK)
    ws = jnp.zeros((NROWS,), jnp.float32).at[pos].set(rwflat)

    xbf = hidden_2d.astype(jnp.bfloat16)
    x_sorted = jnp.take(xbf, gidx, axis=0)

    ws3 = ws.reshape(NT, 1, M)
    act = pl.pallas_call(
        _gmm1_body,
        grid_spec=pltpu.PrefetchScalarGridSpec(
            num_scalar_prefetch=1,
            grid=(K1, NT),
            in_specs=[
                pl.BlockSpec((NROWS, D), lambda k, t, te: (0, 0)),
                pl.BlockSpec((1, F_BLK, D), lambda k, t, te: (te[t], k, 0)),
                pl.BlockSpec((1, F_BLK, D),
                             lambda k, t, te: (te[t], K1 + k, 0)),
                pl.BlockSpec((1, 1, M), lambda k, t, te: (t, 0, 0)),
            ],
            out_specs=pl.BlockSpec((M, F_BLK), lambda k, t, te: (t, k)),
        ),
        out_shape=jax.ShapeDtypeStruct((NROWS, DFF), jnp.bfloat16),
        interpret=_INTERPRET,
    )(te, x_sorted, gate_up_proj, gate_up_proj, ws3)

    y_sorted = pl.pallas_call(
        _gmm2_body,
        grid_spec=pltpu.PrefetchScalarGridSpec(
            num_scalar_prefetch=1,
            grid=(NT,),
            in_specs=[
                pl.BlockSpec((M, DFF), lambda t, te: (t, 0)),
                pl.BlockSpec((1, D, DFF), lambda t, te: (te[t], 0, 0)),
            ],
            out_specs=pl.BlockSpec((M, D), lambda t, te: (t, 0)),
        ),
        out_shape=jax.ShapeDtypeStruct((NROWS, D), jnp.float32),
        interpret=_INTERPRET,
    )(te, act, down_proj)

    pos2 = pos.reshape(T, TOPK)
    y1 = jnp.take(y_sorted, pos2[:, 0], axis=0)
    y2 = jnp.take(y_sorted, pos2[:, 1], axis=0)
    out = h3.reshape(T, D) + y1 + y2
    return out.reshape(B, S, D)
